# baseline (speedup 1.0000x reference)
import jax
import jax.numpy as jnp
from jax import lax
from jax.experimental import pallas as pl
from jax.experimental.pallas import tpu as pltpu
from jax.experimental.pallas import tpu_sc as plsc

_SMOOTH = 1e-5
_C = 19
_NC, _NS, _NL = 2, 16, 16
_NW = _NC * _NS
_CR = 4
_W = 512
_LN2 = 0.6931471805599453


def _log16(x):
    xi = plsc.bitcast(x, jnp.int32)
    e = (xi >> 23) - 127
    m = plsc.bitcast((xi & 0x007FFFFF) | 0x3F800000, jnp.float32)
    big = m > 1.4142135
    m = jnp.where(big, m * 0.5, m)
    e = jnp.where(big, e + 1, e)
    t = (m - 1.0) / (m + 1.0)
    t2 = t * t
    p = 2.0 + t2 * (2.0 / 3.0 + t2 * (2.0 / 5.0 + t2 * (2.0 / 7.0)))
    return e.astype(jnp.float32) * _LN2 + t * p


def _sc_body(lg_hbm, tg_hbm, out_hbm, buf0, buf1, tb0, tb1, accv, sem0, sem1):
    B, _, H, _ = lg_hbm.shape
    cpb = H // _CR
    cpw = (B * cpb) // _NW
    a_coef = 1.0 - _SMOOTH - _SMOOTH / (_C - 1)
    b_coef = _SMOOTH / (_C - 1)
    wid = lax.axis_index("s") * _NC + lax.axis_index("c")
    bufs, tbs, sems = (buf0, buf1), (tb0, tb1), (sem0, sem1)
    lane = lax.broadcasted_iota(jnp.int32, (_NL,), 0)

    def issue(i, q):
        cid = wid * cpw + i
        b = cid // cpb
        r0 = (cid % cpb) * _CR
        for c in range(_C):
            pltpu.async_copy(
                lg_hbm.at[b, c, pl.ds(r0, _CR), :], bufs[q].at[c], sems[q])
        pltpu.async_copy(
            tg_hbm.at[b, 0, pl.ds(r0, _CR), :], tbs[q], sems[q])

    def drain(q):
        for c in range(_C):
            pltpu.make_async_copy(
                lg_hbm.at[0, c, pl.ds(0, _CR), :], bufs[q].at[c],
                sems[q]).wait()
        pltpu.make_async_copy(
            tg_hbm.at[0, 0, pl.ds(0, _CR), :], tbs[q], sems[q]).wait()

    def px16(buf, tbuf, r, w0, acc):
        t16 = tbuf[r, pl.ds(w0, _NL)]
        lgt = plsc.load_gather(buf, [t16, lane * 0 + r, lane + w0])
        rows = [buf[c, r, pl.ds(w0, _NL)] for c in range(_C)]
        while len(rows) > 1:
            nxt = [rows[2 * j] + rows[2 * j + 1] for j in range(len(rows) // 2)]
            if len(rows) % 2:
                nxt.append(rows[-1])
            rows = nxt
        pt = a_coef * lgt + (b_coef * rows[0] + _SMOOTH)
        om = 1.0 - pt
        return acc + om * om * _log16(pt)

    def pair_body(j, acc):
        for p in (0, 1):
            i = j * 2 + p

            @pl.when(i + 1 < cpw)
            def _():
                issue(i + 1, 1 - p)

            drain(p)

            def k_body(k, acc):
                w0 = k * (2 * _NL)
                for r in range(_CR):
                    acc = px16(bufs[p], tbs[p], r, w0, acc)
                    acc = px16(bufs[p], tbs[p], r, w0 + _NL, acc)
                return acc

            acc = lax.fori_loop(0, _W // (2 * _NL), k_body, acc)
        return acc

    issue(0, 0)
    acc = lax.fori_loop(0, cpw // 2, pair_body,
                        jnp.zeros((_NL,), jnp.float32))
    accv[...] = acc
    pltpu.sync_copy(accv, out_hbm.at[wid])


def kernel(logit, target):
    B, C, H, W = logit.shape
    tgt = target.astype(jnp.int32)
    mesh = plsc.VectorSubcoreMesh(core_axis_name="c", subcore_axis_name="s")
    partials = pl.kernel(
        _sc_body,
        out_type=jax.ShapeDtypeStruct((_NW, _NL), jnp.float32),
        mesh=mesh,
        scratch_types=[
            pltpu.VMEM((_C, _CR, _W), jnp.float32),
            pltpu.VMEM((_C, _CR, _W), jnp.float32),
            pltpu.VMEM((_CR, _W), jnp.int32),
            pltpu.VMEM((_CR, _W), jnp.int32),
            pltpu.VMEM((_NL,), jnp.float32),
            pltpu.SemaphoreType.DMA,
            pltpu.SemaphoreType.DMA,
        ],
        compiler_params=pltpu.CompilerParams(needs_layout_passes=False),
    )(logit, tgt)
    return -jnp.sum(partials) / (B * H * W)


# speedup vs baseline: 1.0103x; 1.0103x over previous
import jax
import jax.numpy as jnp
from jax import lax
from jax.experimental import pallas as pl
from jax.experimental.pallas import tpu as pltpu
from jax.experimental.pallas import tpu_sc as plsc

_SMOOTH = 1e-5
_C = 19
_NC, _NS, _NL = 2, 16, 16
_NW = _NC * _NS
_CR = 4
_W = 512
_LN2 = 0.6931471805599453


def _log16(x):
    xi = plsc.bitcast(x, jnp.int32)
    e = (xi >> 23) - 127
    m = plsc.bitcast((xi & 0x007FFFFF) | 0x3F800000, jnp.float32)
    big = m > 1.4142135
    m = jnp.where(big, m * 0.5, m)
    e = jnp.where(big, e + 1, e)
    t = (m - 1.0) / (m + 1.0)
    t2 = t * t
    p = 2.0 + t2 * (2.0 / 3.0 + t2 * (2.0 / 5.0 + t2 * (2.0 / 7.0)))
    return e.astype(jnp.float32) * _LN2 + t * p


def _sc_body(lg_hbm, tg_hbm, out_hbm, buf0, buf1, tb0, tb1, accv, sem0, sem1):
    B, _, H, _ = lg_hbm.shape
    cpb = H // _CR
    cpw = (B * cpb) // _NW
    a_coef = 1.0 - _SMOOTH - _SMOOTH / (_C - 1)
    b_coef = _SMOOTH / (_C - 1)
    wid = lax.axis_index("s") * _NC + lax.axis_index("c")
    bufs, tbs, sems = (buf0, buf1), (tb0, tb1), (sem0, sem1)
    lane = lax.broadcasted_iota(jnp.int32, (_NL,), 0)

    def issue(i, q):
        cid = wid * cpw + i
        b = cid // cpb
        r0 = (cid % cpb) * _CR
        pltpu.async_copy(
            lg_hbm.at[b, :, pl.ds(r0, _CR), :], bufs[q], sems[q])
        pltpu.async_copy(
            tg_hbm.at[b, 0, pl.ds(r0, _CR), :], tbs[q], sems[q])

    def drain(q):
        pltpu.make_async_copy(
            lg_hbm.at[0, :, pl.ds(0, _CR), :], bufs[q], sems[q]).wait()
        pltpu.make_async_copy(
            tg_hbm.at[0, 0, pl.ds(0, _CR), :], tbs[q], sems[q]).wait()

    def px16(buf, tbuf, r, w0, acc):
        t16 = tbuf[r, pl.ds(w0, _NL)]
        lgt = plsc.load_gather(buf, [t16, lane * 0 + r, lane + w0])
        rows = [buf[c, r, pl.ds(w0, _NL)] for c in range(_C)]
        while len(rows) > 1:
            nxt = [rows[2 * j] + rows[2 * j + 1] for j in range(len(rows) // 2)]
            if len(rows) % 2:
                nxt.append(rows[-1])
            rows = nxt
        pt = a_coef * lgt + (b_coef * rows[0] + _SMOOTH)
        om = 1.0 - pt
        return acc + om * om * _log16(pt)

    def pair_body(j, acc):
        for p in (0, 1):
            i = j * 2 + p

            @pl.when(i + 1 < cpw)
            def _():
                issue(i + 1, 1 - p)

            drain(p)

            def k_body(k, acc):
                w0 = k * (2 * _NL)
                for r in range(_CR):
                    acc = px16(bufs[p], tbs[p], r, w0, acc)
                    acc = px16(bufs[p], tbs[p], r, w0 + _NL, acc)
                return acc

            acc = lax.fori_loop(0, _W // (2 * _NL), k_body, acc)
        return acc

    issue(0, 0)
    acc = lax.fori_loop(0, cpw // 2, pair_body,
                        jnp.zeros((_NL,), jnp.float32))
    accv[...] = acc
    pltpu.sync_copy(accv, out_hbm.at[wid])


def kernel(logit, target):
    B, C, H, W = logit.shape
    tgt = target.astype(jnp.int32)
    mesh = plsc.VectorSubcoreMesh(core_axis_name="c", subcore_axis_name="s")
    partials = pl.kernel(
        _sc_body,
        out_type=jax.ShapeDtypeStruct((_NW, _NL), jnp.float32),
        mesh=mesh,
        scratch_types=[
            pltpu.VMEM((_C, _CR, _W), jnp.float32),
            pltpu.VMEM((_C, _CR, _W), jnp.float32),
            pltpu.VMEM((_CR, _W), jnp.int32),
            pltpu.VMEM((_CR, _W), jnp.int32),
            pltpu.VMEM((_NL,), jnp.float32),
            pltpu.SemaphoreType.DMA,
            pltpu.SemaphoreType.DMA,
        ],
        compiler_params=pltpu.CompilerParams(needs_layout_passes=False),
    )(logit, tgt)
    return -jnp.sum(partials) / (B * H * W)
